# baseline (speedup 1.0000x reference)
import functools

import jax
import jax.numpy as jnp
from jax import lax
from jax.experimental import pallas as pl
from jax.experimental.pallas import tpu as pltpu
from jax.experimental.pallas import tpu_sc as plsc

N = 10000
D_IN = 128
D_H = 128
D_OUT = 64
E = 320000

NC = 2
NS = 16
L = 16
NW = NC * NS

NPAD = 10112
RPT = NPAD // NS
NPADD = 10240
RPTD = NPADD // NS
CHD = 64
NCHUNKD = 160
CH = 32
EPW = 10240
NCHUNK = EPW // CH
EPAD = EPW * NW
HROWS = 10240
HHALF = HROWS // 2
NB = 2
NID = 4
NCHUNKA = EPAD // (NS * CH)

_MESH = plsc.VectorSubcoreMesh(core_axis_name="c", subcore_axis_name="s")


@functools.partial(
    pl.kernel,
    out_type=jax.ShapeDtypeStruct((NC, 2, NPADD), jnp.float32),
    mesh=_MESH,
    scratch_types=[
        pltpu.VMEM((NCHUNKD, 2 * CHD), jnp.int32),
        pltpu.VMEM((CHD,), jnp.float32),
        pltpu.VMEM_SHARED((NPADD,), jnp.float32),
        pltpu.VMEM_SHARED((NPADD,), jnp.float32),
    ],
)
def _deg_call(edges_hbm, zeros_hbm, out_hbm, eidx, ones_v, acc_s, acc_d):
    cid = lax.axis_index("c")
    sid = lax.axis_index("s")
    w = cid * NS + sid
    pltpu.sync_copy(zeros_hbm, acc_s.at[pl.ds(sid * RPTD, RPTD)])
    pltpu.sync_copy(zeros_hbm, acc_d.at[pl.ds(sid * RPTD, RPTD)])
    pltpu.sync_copy(edges_hbm.at[w], eidx)
    for i in range(CHD // L):
        ones_v[pl.ds(i * L, L)] = jnp.ones((L,), jnp.float32)
    plsc.subcore_barrier()

    def chunk(j, carry):
        pltpu.sync_copy(ones_v, acc_s.at[eidx.at[j, pl.ds(0, CHD)]], add=True)
        pltpu.sync_copy(ones_v, acc_d.at[eidx.at[j, pl.ds(CHD, CHD)]], add=True)
        return carry

    lax.fori_loop(0, NCHUNKD, chunk, 0)
    plsc.subcore_barrier()
    pltpu.sync_copy(acc_s.at[pl.ds(sid * RPTD, RPTD)],
                    out_hbm.at[cid, 0, pl.ds(sid * RPTD, RPTD)])
    pltpu.sync_copy(acc_d.at[pl.ds(sid * RPTD, RPTD)],
                    out_hbm.at[cid, 1, pl.ds(sid * RPTD, RPTD)])


def _make_agg(d):
    @functools.partial(
        pl.kernel,
        out_type=jax.ShapeDtypeStruct((NC, NPAD, d), jnp.float32),
        mesh=_MESH,
        scratch_types=[
            [pltpu.VMEM((1, 2 * CH), jnp.int32) for _ in range(NID)],
            [pltpu.VMEM((CH, d), jnp.float32) for _ in range(NB)],
            pltpu.VMEM_SHARED((HHALF, d), jnp.float32),
            pltpu.VMEM_SHARED((NPAD, d), jnp.float32),
            [pltpu.SemaphoreType.DMA for _ in range(NID)],
            [pltpu.SemaphoreType.DMA for _ in range(NB)],
            [pltpu.SemaphoreType.DMA for _ in range(NB)],
        ],
    )
    def agg(h_hbm, edges_hbm, zeros_hbm, out_hbm, ibufs, rows, tab, acc,
            isems, gsems, ssems):
        cid = lax.axis_index("c")
        sid = lax.axis_index("s")
        pltpu.sync_copy(zeros_hbm, acc.at[pl.ds(sid * RPT, RPT)])
        spt = HHALF // NS
        pltpu.sync_copy(h_hbm.at[pl.ds(cid * HHALF + sid * spt, spt)],
                        tab.at[pl.ds(sid * spt, spt)])
        plsc.subcore_barrier()

        def fire_idx(j, b):
            pltpu.async_copy(edges_hbm.at[cid, sid, j], ibufs[b], isems[b])

        def wait_idx(j, b):
            pltpu.make_async_copy(edges_hbm.at[cid, sid, j], ibufs[b], isems[b]).wait()

        def launch_gather(b, r):
            pltpu.async_copy(tab.at[ibufs[b].at[0, pl.ds(0, CH)]], rows[r], gsems[r])

        def wait_gather(b, r):
            pltpu.make_async_copy(tab.at[ibufs[b].at[0, pl.ds(0, CH)]],
                                  rows[r], gsems[r]).wait()

        def fire_scatter(b, r):
            pltpu.async_copy(rows[r], acc.at[ibufs[b].at[0, pl.ds(CH, CH)]],
                             ssems[r], add=True)

        def drain_scatter(b, r):
            pltpu.make_async_copy(rows[r], acc.at[ibufs[b].at[0, pl.ds(CH, CH)]],
                                  ssems[r]).wait()

        for k in range(NID - 1):
            fire_idx(k, k)
        wait_idx(0, 0)
        wait_idx(1, 1)
        launch_gather(0, 0)

        def step(m, carry):
            for u in range(NID):
                j = m * NID + u
                jr = j + 2

                @pl.when(jr < NCHUNKA)
                def _():
                    wait_idx(jr, (u + 2) % NID)

                @pl.when(j >= 1)
                def _():
                    drain_scatter((u + NID - 1) % NID, (u + 1) % NB)

                jf = j + NID - 1

                @pl.when(jf < NCHUNKA)
                def _():
                    fire_idx(jf, (u + NID - 1) % NID)

                jl = j + 1

                @pl.when(jl < NCHUNKA)
                def _():
                    launch_gather((u + 1) % NID, (u + 1) % NB)

                wait_gather(u, u % NB)
                fire_scatter(u, u % NB)
            return carry

        lax.fori_loop(0, NCHUNKA // NID, step, 0)
        drain_scatter((NCHUNKA - 1) % NID, (NCHUNKA - 1) % NB)
        plsc.subcore_barrier()
        pltpu.sync_copy(acc.at[pl.ds(sid * RPT, RPT)],
                        out_hbm.at[cid, pl.ds(sid * RPT, RPT)])

    return agg


_agg_h = _make_agg(D_H)


def _norm_body(c_ref, o_ref):
    s = c_ref[0] + c_ref[1]
    o_ref[...] = lax.rsqrt(jnp.maximum(s, 1.0))


def _norm_call(counts):
    return pl.pallas_call(
        _norm_body,
        out_shape=jax.ShapeDtypeStruct((2, NPADD), jnp.float32),
    )(counts)


RB = 2000


def _mm1_body(x_ref, w_ref, ns_ref, o_ref):
    o_ref[...] = jnp.dot(x_ref[...], w_ref[...],
                         preferred_element_type=jnp.float32) * ns_ref[...]


def _mm1_call(x, w1, norm_src):
    return pl.pallas_call(
        _mm1_body,
        grid=(N // RB,),
        in_specs=[
            pl.BlockSpec((RB, D_IN), lambda i: (i, 0)),
            pl.BlockSpec((D_IN, D_H), lambda i: (0, 0)),
            pl.BlockSpec((RB, 1), lambda i: (i, 0)),
        ],
        out_specs=pl.BlockSpec((RB, D_H), lambda i: (i, 0)),
        out_shape=jax.ShapeDtypeStruct((N, D_H), jnp.float32),
    )(x, w1, norm_src)


def _mid_body(p_ref, nd_ref, ns_ref, b1_ref, o_ref):
    h = p_ref[0] + p_ref[1]
    h = jnp.maximum(h * nd_ref[...] + b1_ref[...], 0.0)
    o_ref[...] = h * ns_ref[...]


def _mid_call(p, norm_dst, norm_src, b1):
    return pl.pallas_call(
        _mid_body,
        grid=(N // RB,),
        in_specs=[
            pl.BlockSpec((2, RB, D_H), lambda i: (0, i, 0)),
            pl.BlockSpec((RB, 1), lambda i: (i, 0)),
            pl.BlockSpec((RB, 1), lambda i: (i, 0)),
            pl.BlockSpec((1, D_H), lambda i: (0, 0)),
        ],
        out_specs=pl.BlockSpec((RB, D_H), lambda i: (i, 0)),
        out_shape=jax.ShapeDtypeStruct((N, D_H), jnp.float32),
    )(p, norm_dst, norm_src, b1)


def _fin_body(q_ref, nd_ref, w2_ref, b2_ref, o_ref):
    t = (q_ref[0] + q_ref[1]) * nd_ref[...]
    z = jnp.dot(t, w2_ref[...], preferred_element_type=jnp.float32) + b2_ref[...]
    m = jnp.max(z, axis=1, keepdims=True)
    zs = z - m
    o_ref[...] = zs - jnp.log(jnp.sum(jnp.exp(zs), axis=1, keepdims=True))


def _fin_call(q, norm_dst, w2, b2):
    return pl.pallas_call(
        _fin_body,
        grid=(N // RB,),
        in_specs=[
            pl.BlockSpec((2, RB, D_H), lambda i: (0, i, 0)),
            pl.BlockSpec((RB, 1), lambda i: (i, 0)),
            pl.BlockSpec((D_H, D_OUT), lambda i: (0, 0)),
            pl.BlockSpec((1, D_OUT), lambda i: (0, 0)),
        ],
        out_specs=pl.BlockSpec((RB, D_OUT), lambda i: (i, 0)),
        out_shape=jax.ShapeDtypeStruct((N, D_OUT), jnp.float32),
    )(q, norm_dst, w2, b2)


def kernel(in_feat, edge_index, W1, b1, W2, b2):
    src = edge_index[0].astype(jnp.int32)
    dst = edge_index[1].astype(jnp.int32)
    pad = jnp.full((EPAD - E,), N, dtype=jnp.int32)
    src_p = jnp.concatenate([src, pad])
    dst_p = jnp.concatenate([dst, pad])
    edges_deg = jnp.concatenate(
        [src_p.reshape(NW, NCHUNKD, CHD), dst_p.reshape(NW, NCHUNKD, CHD)], axis=-1)
    cores = []
    for c in range(NC):
        t = src_p - c * HHALF
        oob = (t < 0) | (t >= HHALF)
        s_r = jnp.where(oob, src_p & 4095, t)
        d_r = jnp.where(oob, N + (dst_p & 63), dst_p)
        cores.append(jnp.concatenate(
            [s_r.reshape(NS, NCHUNKA, 1, CH), d_r.reshape(NS, NCHUNKA, 1, CH)],
            axis=-1))
    edges_agg = jnp.stack(cores)
    zeros1 = jnp.zeros((RPTD,), jnp.float32)
    zeros_h = jnp.zeros((RPT, D_H), jnp.float32)

    counts = _deg_call(edges_deg, zeros1)
    norms = _norm_call(counts)
    norm_src = norms[0, :N].reshape(N, 1)
    norm_dst = norms[1, :N].reshape(N, 1)

    hs = _mm1_call(in_feat, W1, norm_src)
    hs_pad = jnp.concatenate([hs, jnp.zeros((HROWS - N, D_H), jnp.float32)])
    p1 = _agg_h(hs_pad, edges_agg, zeros_h)
    p1 = p1[:, :N]

    g2 = _mid_call(p1, norm_dst, norm_src, b1.reshape(1, D_H))
    g2_pad = jnp.concatenate([g2, jnp.zeros((HROWS - N, D_H), jnp.float32)])
    p2 = _agg_h(g2_pad, edges_agg, zeros_h)
    p2 = p2[:, :N]

    return _fin_call(p2, norm_dst, W2, b2.reshape(1, D_OUT))


# speedup vs baseline: 1.0535x; 1.0535x over previous
import functools

import jax
import jax.numpy as jnp
from jax import lax
from jax.experimental import pallas as pl
from jax.experimental.pallas import tpu as pltpu
from jax.experimental.pallas import tpu_sc as plsc

N = 10000
D_IN = 128
D_H = 128
D_OUT = 64
E = 320000

NC = 2
NS = 16
L = 16
NW = NC * NS

NPAD = 10112
RPT = NPAD // NS
NPADD = 10240
RPTD = NPADD // NS
CHD = 64
NCHUNKD = 160
CH = 32
EPW = 10240
NCHUNK = EPW // CH
EPAD = EPW * NW
HROWS = 10240
HHALF = HROWS // 2
NB = 2
NID = 4
NCHUNKA = EPAD // (NS * CH)

_MESH = plsc.VectorSubcoreMesh(core_axis_name="c", subcore_axis_name="s")


@functools.partial(
    pl.kernel,
    out_type=jax.ShapeDtypeStruct((NC, 2, NPADD), jnp.float32),
    mesh=_MESH,
    scratch_types=[
        pltpu.VMEM((NCHUNKD, 2 * CHD), jnp.int32),
        pltpu.VMEM((CHD,), jnp.float32),
        pltpu.VMEM_SHARED((NPADD,), jnp.float32),
        pltpu.VMEM_SHARED((NPADD,), jnp.float32),
    ],
)
def _deg_call(edges_hbm, zeros_hbm, out_hbm, eidx, ones_v, acc_s, acc_d):
    cid = lax.axis_index("c")
    sid = lax.axis_index("s")
    w = cid * NS + sid
    pltpu.sync_copy(zeros_hbm, acc_s.at[pl.ds(sid * RPTD, RPTD)])
    pltpu.sync_copy(zeros_hbm, acc_d.at[pl.ds(sid * RPTD, RPTD)])
    pltpu.sync_copy(edges_hbm.at[w], eidx)
    for i in range(CHD // L):
        ones_v[pl.ds(i * L, L)] = jnp.ones((L,), jnp.float32)
    plsc.subcore_barrier()

    def chunk(j, carry):
        pltpu.sync_copy(ones_v, acc_s.at[eidx.at[j, pl.ds(0, CHD)]], add=True)
        pltpu.sync_copy(ones_v, acc_d.at[eidx.at[j, pl.ds(CHD, CHD)]], add=True)
        return carry

    lax.fori_loop(0, NCHUNKD, chunk, 0)
    plsc.subcore_barrier()
    pltpu.sync_copy(acc_s.at[pl.ds(sid * RPTD, RPTD)],
                    out_hbm.at[cid, 0, pl.ds(sid * RPTD, RPTD)])
    pltpu.sync_copy(acc_d.at[pl.ds(sid * RPTD, RPTD)],
                    out_hbm.at[cid, 1, pl.ds(sid * RPTD, RPTD)])


def _make_agg(d):
    @functools.partial(
        pl.kernel,
        out_type=jax.ShapeDtypeStruct((NC, NPAD, d), jnp.float32),
        mesh=_MESH,
        scratch_types=[
            [pltpu.VMEM((1, CH), jnp.int32) for _ in range(NID)],
            [pltpu.VMEM((1, CH), jnp.int32) for _ in range(NID)],
            [pltpu.VMEM((CH, d), jnp.float32) for _ in range(NB)],
            pltpu.VMEM_SHARED((HHALF, d), jnp.float32),
            pltpu.VMEM_SHARED((NPAD, d), jnp.float32),
            [pltpu.SemaphoreType.DMA for _ in range(NID)],
            [pltpu.SemaphoreType.DMA for _ in range(NB)],
        ],
    )
    def agg(h_hbm, src_hbm, dst_hbm, zeros_hbm, out_hbm, sbufs, dbufs, rows, tab, acc, isems, gsems):
        cid = lax.axis_index("c")
        sid = lax.axis_index("s")
        base = jnp.zeros((L,), jnp.int32) + cid * HHALF
        pltpu.sync_copy(zeros_hbm, acc.at[pl.ds(sid * RPT, RPT)])
        spt = HHALF // NS
        pltpu.sync_copy(h_hbm.at[pl.ds(cid * HHALF + sid * spt, spt)],
                        tab.at[pl.ds(sid * spt, spt)])
        plsc.subcore_barrier()

        def fire_idx(j, b):
            pltpu.async_copy(src_hbm.at[cid, sid, j], sbufs[b], isems[b])
            pltpu.async_copy(dst_hbm.at[cid, sid, j], dbufs[b], isems[b])

        def remap(b):
            ib = ibufs[b]
            for k in range(CH // L):
                sv = ib[0, pl.ds(k * L, L)]
                dv = ib[0, pl.ds(CH + k * L, L)]
                t = sv - base
                oob = (t < 0) | (t >= HHALF)
                ib[0, pl.ds(k * L, L)] = jnp.where(oob, sv & 4095, t)
                ib[0, pl.ds(CH + k * L, L)] = jnp.where(oob, N + (dv & 63), dv)

        def launch_gather(b, r):
            pltpu.async_copy(tab.at[sbufs[b].at[0]], rows[r], gsems[r])

        def wait_gather(b, r):
            pltpu.make_async_copy(tab.at[sbufs[b].at[0]], rows[r], gsems[r]).wait()

        def wait_idx(j, b):
            pltpu.make_async_copy(src_hbm.at[cid, sid, j], sbufs[b], isems[b]).wait()
            pltpu.make_async_copy(dst_hbm.at[cid, sid, j], dbufs[b], isems[b]).wait()

        for k in range(NID - 1):
            fire_idx(k, k)
        wait_idx(0, 0)
        wait_idx(1, 1)
        launch_gather(0, 0)

        def step(m, carry):
            for u in range(NID):
                j = m * NID + u
                jf = j + NID - 1

                @pl.when(jf < NCHUNKA)
                def _():
                    fire_idx(jf, (u + NID - 1) % NID)

                jr = j + 2

                @pl.when(jr < NCHUNKA)
                def _():
                    wait_idx(jr, (u + 2) % NID)

                jl = j + 1

                @pl.when(jl < NCHUNKA)
                def _():
                    launch_gather((u + 1) % NID, (u + 1) % NB)

                wait_gather(u, u % NB)
                pltpu.sync_copy(rows[u % NB],
                                acc.at[dbufs[u].at[0]], add=True)
            return carry

        lax.fori_loop(0, NCHUNKA // NID, step, 0)
        plsc.subcore_barrier()
        pltpu.sync_copy(acc.at[pl.ds(sid * RPT, RPT)],
                        out_hbm.at[cid, pl.ds(sid * RPT, RPT)])

    return agg


_agg_h = _make_agg(D_H)


def _norm_body(c_ref, o_ref):
    s = c_ref[0] + c_ref[1]
    o_ref[...] = lax.rsqrt(jnp.maximum(s, 1.0))


def _norm_call(counts):
    return pl.pallas_call(
        _norm_body,
        out_shape=jax.ShapeDtypeStruct((2, NPADD), jnp.float32),
    )(counts)


RB = 2000


def _mm1_body(x_ref, w_ref, ns_ref, o_ref):
    o_ref[...] = jnp.dot(x_ref[...], w_ref[...],
                         preferred_element_type=jnp.float32) * ns_ref[...]


def _mm1_call(x, w1, norm_src):
    return pl.pallas_call(
        _mm1_body,
        grid=(N // RB,),
        in_specs=[
            pl.BlockSpec((RB, D_IN), lambda i: (i, 0)),
            pl.BlockSpec((D_IN, D_H), lambda i: (0, 0)),
            pl.BlockSpec((RB, 1), lambda i: (i, 0)),
        ],
        out_specs=pl.BlockSpec((RB, D_H), lambda i: (i, 0)),
        out_shape=jax.ShapeDtypeStruct((N, D_H), jnp.float32),
    )(x, w1, norm_src)


def _mid_body(p_ref, nd_ref, ns_ref, b1_ref, o_ref):
    h = p_ref[0] + p_ref[1]
    h = jnp.maximum(h * nd_ref[...] + b1_ref[...], 0.0)
    o_ref[...] = h * ns_ref[...]


def _mid_call(p, norm_dst, norm_src, b1):
    return pl.pallas_call(
        _mid_body,
        grid=(N // RB,),
        in_specs=[
            pl.BlockSpec((2, RB, D_H), lambda i: (0, i, 0)),
            pl.BlockSpec((RB, 1), lambda i: (i, 0)),
            pl.BlockSpec((RB, 1), lambda i: (i, 0)),
            pl.BlockSpec((1, D_H), lambda i: (0, 0)),
        ],
        out_specs=pl.BlockSpec((RB, D_H), lambda i: (i, 0)),
        out_shape=jax.ShapeDtypeStruct((N, D_H), jnp.float32),
    )(p, norm_dst, norm_src, b1)


def _fin_body(q_ref, nd_ref, w2_ref, b2_ref, o_ref):
    t = (q_ref[0] + q_ref[1]) * nd_ref[...]
    z = jnp.dot(t, w2_ref[...], preferred_element_type=jnp.float32) + b2_ref[...]
    m = jnp.max(z, axis=1, keepdims=True)
    zs = z - m
    o_ref[...] = zs - jnp.log(jnp.sum(jnp.exp(zs), axis=1, keepdims=True))


def _fin_call(q, norm_dst, w2, b2):
    return pl.pallas_call(
        _fin_body,
        grid=(N // RB,),
        in_specs=[
            pl.BlockSpec((2, RB, D_H), lambda i: (0, i, 0)),
            pl.BlockSpec((RB, 1), lambda i: (i, 0)),
            pl.BlockSpec((D_H, D_OUT), lambda i: (0, 0)),
            pl.BlockSpec((1, D_OUT), lambda i: (0, 0)),
        ],
        out_specs=pl.BlockSpec((RB, D_OUT), lambda i: (i, 0)),
        out_shape=jax.ShapeDtypeStruct((N, D_OUT), jnp.float32),
    )(q, norm_dst, w2, b2)


def kernel(in_feat, edge_index, W1, b1, W2, b2):
    src = edge_index[0].astype(jnp.int32)
    dst = edge_index[1].astype(jnp.int32)
    pad = jnp.full((EPAD - E,), N, dtype=jnp.int32)
    src_p = jnp.concatenate([src, pad])
    dst_p = jnp.concatenate([dst, pad])
    edges_deg = jnp.concatenate(
        [src_p.reshape(NW, NCHUNKD, CHD), dst_p.reshape(NW, NCHUNKD, CHD)], axis=-1)
    cores = []
    for c in range(NC):
        t = src_p - c * HHALF
        oob = (t < 0) | (t >= HHALF)
        s_r = jnp.where(oob, src_p & 4095, t)
        d_r = jnp.where(oob, N + (dst_p & 63), dst_p)
        cores.append((s_r.reshape(NS, NCHUNKA, 1, CH), d_r.reshape(NS, NCHUNKA, 1, CH)))
    src_agg = jnp.stack([c[0] for c in cores])
    dst_agg = jnp.stack([c[1] for c in cores])
    zeros1 = jnp.zeros((RPTD,), jnp.float32)
    zeros_h = jnp.zeros((RPT, D_H), jnp.float32)

    counts = _deg_call(edges_deg, zeros1)
    norms = _norm_call(counts)
    norm_src = norms[0, :N].reshape(N, 1)
    norm_dst = norms[1, :N].reshape(N, 1)

    hs = _mm1_call(in_feat, W1, norm_src)
    hs_pad = jnp.concatenate([hs, jnp.zeros((HROWS - N, D_H), jnp.float32)])
    p1 = _agg_h(hs_pad, src_agg, dst_agg, zeros_h)
    p1 = p1[:, :N]

    g2 = _mid_call(p1, norm_dst, norm_src, b1.reshape(1, D_H))
    g2_pad = jnp.concatenate([g2, jnp.zeros((HROWS - N, D_H), jnp.float32)])
    p2 = _agg_h(g2_pad, src_agg, dst_agg, zeros_h)
    p2 = p2[:, :N]

    return _fin_call(p2, norm_dst, W2, b2.reshape(1, D_OUT))
